# baseline (speedup 1.0000x reference)
import functools

import jax
import jax.numpy as jnp
from jax.experimental import pallas as pl

_K = 20
_BLK = 128


def _knn_kernel(q_ref, aT_ref, allhl_ref, A_ref, C_ref, g_ref, s_ref):
    blk = q_ref.shape[1]
    nblk = pl.program_id(1)
    q = q_ref[0]
    aT = aT_ref[0]
    allhl = allhl_ref[0]
    hpp = 0.5 * jnp.sum(aT * aT, axis=0, keepdims=True)
    d = jnp.dot(q, aT, preferred_element_type=jnp.float32) - hpp

    A = A_ref[...]
    C = C_ref[...]
    a = jnp.dot(q, A, preferred_element_type=jnp.float32)

    n = aT.shape[1]
    col = jax.lax.broadcasted_iota(jnp.int32, (blk, n), 1)
    row = jax.lax.broadcasted_iota(jnp.int32, (blk, n), 0) + nblk * blk
    d = jnp.where(col == row, -1e30, d)
    y0 = a + jnp.dot(q, C, preferred_element_type=jnp.float32)
    acc1 = y0
    acc2 = y0 * y0
    parts = [q]
    for it in range(_K - 1):
        m = jnp.max(d, axis=1, keepdims=True)
        oh = d >= m
        nbr6 = jnp.dot(oh.astype(jnp.bfloat16), allhl,
                       preferred_element_type=jnp.float32)
        nbr = nbr6[:, 0:3] + nbr6[:, 3:6]
        if it < _K - 2:
            d = jnp.where(oh, -1e30, d)
        y = a + jnp.dot(nbr, C, preferred_element_type=jnp.float32)
        acc1 = acc1 + y
        acc2 = acc2 + y * y
        parts.append(nbr)
    parts.append(jnp.zeros((blk, 64 - 3 * _K), jnp.float32))
    g_ref[0, 0] = jnp.concatenate(parts, axis=1)

    r0 = jnp.sum(acc1, axis=0, keepdims=True)
    r1 = jnp.sum(acc2, axis=0, keepdims=True)
    s_ref[0, 0] = jnp.concatenate(
        [r0, r1, jnp.zeros((6, 32), jnp.float32)], axis=0)


def _apply_kernel(q_ref, g_ref, A_ref, C_ref, s_ref, gb_ref, o_ref, *, m_tot):
    blk = q_ref.shape[1]
    q = q_ref[0]
    G = g_ref[0, 0]
    a = jnp.dot(q, A_ref[...], preferred_element_type=jnp.float32)
    C = C_ref[...]
    s2 = jnp.sum(s_ref[...], axis=0)
    mean = s2[0:1, :] * (1.0 / m_tot)
    var = s2[1:2, :] * (1.0 / m_tot) - mean * mean
    invstd = jax.lax.rsqrt(var + 1e-5)
    scale = gb_ref[0:1, :] * invstd
    shift = gb_ref[1:2, :] - mean * scale
    best = jnp.full((blk, 32), -jnp.inf, jnp.float32)
    for j in range(_K):
        z = G[:, 3 * j:3 * j + 3]
        y = a + jnp.dot(z, C, preferred_element_type=jnp.float32)
        v = y * scale + shift
        h = v * jnp.clip(v + 3.0, 0.0, 6.0) * (1.0 / 6.0)
        best = jnp.maximum(best, h)
    o_ref[0] = jnp.transpose(best)


def kernel(xyz, W, gamma, beta):
    B, N, _ = xyz.shape
    blk = _BLK if N % _BLK == 0 else N
    nb = N // blk
    xyzT = jnp.transpose(xyz, (0, 2, 1))
    xyz_hi = xyz.astype(jnp.bfloat16)
    xyz_lo = (xyz - xyz_hi.astype(jnp.float32)).astype(jnp.bfloat16)
    xyz_hl = jnp.concatenate([xyz_hi, xyz_lo], axis=-1)
    A = jnp.transpose(W[:, :3] - W[:, 3:], (1, 0))
    C = jnp.transpose(W[:, 3:], (1, 0))
    grid = (B, nb)

    G, S = pl.pallas_call(
        _knn_kernel,
        grid=grid,
        in_specs=[
            pl.BlockSpec((1, blk, 3), lambda b, n: (b, n, 0)),
            pl.BlockSpec((1, 3, N), lambda b, n: (b, 0, 0)),
            pl.BlockSpec((1, N, 6), lambda b, n: (b, 0, 0)),
            pl.BlockSpec((3, 32), lambda b, n: (0, 0)),
            pl.BlockSpec((3, 32), lambda b, n: (0, 0)),
        ],
        out_specs=[
            pl.BlockSpec((1, 1, blk, 64), lambda b, n: (b, n, 0, 0)),
            pl.BlockSpec((1, 1, 8, 32), lambda b, n: (b, n, 0, 0)),
        ],
        out_shape=[
            jax.ShapeDtypeStruct((B, nb, blk, 64), jnp.float32),
            jax.ShapeDtypeStruct((B, nb, 8, 32), jnp.float32),
        ],
    )(xyz, xyzT, xyz_hl, A, C)

    Sf = jnp.reshape(S, (B * nb, 8, 32))
    gb = jnp.stack([gamma, beta], axis=0)

    out = pl.pallas_call(
        functools.partial(_apply_kernel, m_tot=float(B * N * _K)),
        grid=grid,
        in_specs=[
            pl.BlockSpec((1, blk, 3), lambda b, n: (b, n, 0)),
            pl.BlockSpec((1, 1, blk, 64), lambda b, n: (b, n, 0, 0)),
            pl.BlockSpec((3, 32), lambda b, n: (0, 0)),
            pl.BlockSpec((3, 32), lambda b, n: (0, 0)),
            pl.BlockSpec((B * nb, 8, 32), lambda b, n: (0, 0, 0)),
            pl.BlockSpec((2, 32), lambda b, n: (0, 0)),
        ],
        out_specs=pl.BlockSpec((1, 32, blk), lambda b, n: (b, 0, n)),
        out_shape=jax.ShapeDtypeStruct((B, 32, N), jnp.float32),
    )(xyz, G, A, C, Sf, gb)

    return (xyz, out)


# speedup vs baseline: 1.3855x; 1.3855x over previous
import functools

import jax
import jax.numpy as jnp
from jax.experimental import pallas as pl

_K = 20
_BLK = 256


def _knn_kernel(q_ref, aT_ref, allhl_ref, A_ref, C_ref, g_ref, s_ref):
    blk = q_ref.shape[1]
    nblk = pl.program_id(1)
    q = q_ref[0]
    aT = aT_ref[0]
    allhl = allhl_ref[0]
    hpp = 0.5 * jnp.sum(aT * aT, axis=0, keepdims=True)
    d = jnp.dot(q, aT, preferred_element_type=jnp.float32) - hpp

    A = A_ref[...]
    C = C_ref[...]
    a = jnp.dot(q, A, preferred_element_type=jnp.float32)

    n = aT.shape[1]
    col = jax.lax.broadcasted_iota(jnp.int32, (blk, n), 1)
    row = jax.lax.broadcasted_iota(jnp.int32, (blk, n), 0) + nblk * blk
    d = jnp.where(col == row, -1e30, d)
    y0 = a + jnp.dot(q, C, preferred_element_type=jnp.float32)
    acc1 = y0
    acc2 = y0 * y0
    parts = [q]
    for it in range(_K - 1):
        m = jnp.max(d, axis=1, keepdims=True)
        oh = d >= m
        nbr6 = jnp.dot(oh.astype(jnp.bfloat16), allhl,
                       preferred_element_type=jnp.float32)
        nbr = nbr6[:, 0:3] + nbr6[:, 3:6]
        if it < _K - 2:
            d = jnp.where(oh, -1e30, d)
        y = a + jnp.dot(nbr, C, preferred_element_type=jnp.float32)
        acc1 = acc1 + y
        acc2 = acc2 + y * y
        parts.append(nbr)
    parts.append(jnp.zeros((blk, 64 - 3 * _K), jnp.float32))
    g_ref[0, 0] = jnp.concatenate(parts, axis=1)

    r0 = jnp.sum(acc1, axis=0, keepdims=True)
    r1 = jnp.sum(acc2, axis=0, keepdims=True)
    s_ref[0, 0] = jnp.concatenate(
        [r0, r1, jnp.zeros((6, 32), jnp.float32)], axis=0)


def _apply_kernel(q_ref, g_ref, A_ref, C_ref, s_ref, gb_ref, o_ref, *, m_tot):
    blk = q_ref.shape[1]
    q = q_ref[0]
    G = g_ref[0, 0]
    a = jnp.dot(q, A_ref[...], preferred_element_type=jnp.float32)
    C = C_ref[...]
    s2 = jnp.sum(s_ref[...], axis=0)
    mean = s2[0:1, :] * (1.0 / m_tot)
    var = s2[1:2, :] * (1.0 / m_tot) - mean * mean
    invstd = jax.lax.rsqrt(var + 1e-5)
    scale = gb_ref[0:1, :] * invstd
    shift = gb_ref[1:2, :] - mean * scale
    best = jnp.full((blk, 32), -jnp.inf, jnp.float32)
    for j in range(_K):
        z = G[:, 3 * j:3 * j + 3]
        y = a + jnp.dot(z, C, preferred_element_type=jnp.float32)
        v = y * scale + shift
        h = v * jnp.clip(v + 3.0, 0.0, 6.0) * (1.0 / 6.0)
        best = jnp.maximum(best, h)
    o_ref[0] = jnp.transpose(best)


def kernel(xyz, W, gamma, beta):
    B, N, _ = xyz.shape
    blk = _BLK if N % _BLK == 0 else N
    nb = N // blk
    xyzT = jnp.transpose(xyz, (0, 2, 1))
    xyz_hi = xyz.astype(jnp.bfloat16)
    xyz_lo = (xyz - xyz_hi.astype(jnp.float32)).astype(jnp.bfloat16)
    xyz_hl = jnp.concatenate([xyz_hi, xyz_lo], axis=-1)
    A = jnp.transpose(W[:, :3] - W[:, 3:], (1, 0))
    C = jnp.transpose(W[:, 3:], (1, 0))
    grid = (B, nb)

    G, S = pl.pallas_call(
        _knn_kernel,
        grid=grid,
        in_specs=[
            pl.BlockSpec((1, blk, 3), lambda b, n: (b, n, 0)),
            pl.BlockSpec((1, 3, N), lambda b, n: (b, 0, 0)),
            pl.BlockSpec((1, N, 6), lambda b, n: (b, 0, 0)),
            pl.BlockSpec((3, 32), lambda b, n: (0, 0)),
            pl.BlockSpec((3, 32), lambda b, n: (0, 0)),
        ],
        out_specs=[
            pl.BlockSpec((1, 1, blk, 64), lambda b, n: (b, n, 0, 0)),
            pl.BlockSpec((1, 1, 8, 32), lambda b, n: (b, n, 0, 0)),
        ],
        out_shape=[
            jax.ShapeDtypeStruct((B, nb, blk, 64), jnp.float32),
            jax.ShapeDtypeStruct((B, nb, 8, 32), jnp.float32),
        ],
    )(xyz, xyzT, xyz_hl, A, C)

    Sf = jnp.reshape(S, (B * nb, 8, 32))
    gb = jnp.stack([gamma, beta], axis=0)

    out = pl.pallas_call(
        functools.partial(_apply_kernel, m_tot=float(B * N * _K)),
        grid=grid,
        in_specs=[
            pl.BlockSpec((1, blk, 3), lambda b, n: (b, n, 0)),
            pl.BlockSpec((1, 1, blk, 64), lambda b, n: (b, n, 0, 0)),
            pl.BlockSpec((3, 32), lambda b, n: (0, 0)),
            pl.BlockSpec((3, 32), lambda b, n: (0, 0)),
            pl.BlockSpec((B * nb, 8, 32), lambda b, n: (0, 0, 0)),
            pl.BlockSpec((2, 32), lambda b, n: (0, 0)),
        ],
        out_specs=pl.BlockSpec((1, 32, blk), lambda b, n: (b, 0, n)),
        out_shape=jax.ShapeDtypeStruct((B, 32, N), jnp.float32),
    )(xyz, G, A, C, Sf, gb)

    return (xyz, out)
